# Initial kernel scaffold; baseline (speedup 1.0000x reference)
import jax
import jax.numpy as jnp
from jax.experimental import pallas as pl


def kernel(g, features, W1, b1, W2, b2):
    raise NotImplementedError("write your pallas kernel here")


# speedup vs baseline: 5.2335x; 5.2335x over previous
import functools

import jax
import jax.numpy as jnp
from jax import lax
from jax.experimental import pallas as pl
from jax.experimental.pallas import tpu as pltpu
from jax.experimental.pallas import tpu_sc as plsc

N = 10000
E = 320000
D = 128

NC = 2
NS = 16
NW = NC * NS
EPW = E // NW
K = 128
NCHUNK = -(-EPW // K)
EPAD = NCHUNK * K
R0 = 624
R15 = N - 15 * R0
DEGW = 128

_MESH = plsc.VectorSubcoreMesh(core_axis_name="c", subcore_axis_name="s")


@functools.partial(
    pl.kernel, mesh=_MESH,
    out_type=[jax.ShapeDtypeStruct((NC, N, D), jnp.float32)],
    scratch_types=[
        pltpu.VMEM_SHARED((N + 16, D), jnp.float32),
        pltpu.VMEM((NCHUNK, K), jnp.int32),
        pltpu.VMEM((NCHUNK, K), jnp.int32),
        pltpu.VMEM((K, D), jnp.float32),
        pltpu.SemaphoreType.DMA,
    ])
def _sc_agg(x_hbm, src_hbm, dst_hbm, out_hbm, acc_sh, sidx, didx, rows,
            sem):
    cid = lax.axis_index("c")
    sid = lax.axis_index("s")
    wid = cid * NS + sid
    row0 = sid * R0
    last = sid == NS - 1

    def zrow(r, carry):
        for j in range(D // 16):
            rows[r, pl.ds(j * 16, 16)] = jnp.zeros((16,), jnp.float32)
        return carry
    lax.fori_loop(0, K, zrow, 0)

    @pl.when(jnp.logical_and(cid == 0, jnp.logical_not(last)))
    def _():
        pltpu.sync_copy(x_hbm.at[pl.ds(row0, R0)],
                        acc_sh.at[pl.ds(row0, R0)])

    @pl.when(jnp.logical_and(cid == 0, last))
    def _():
        pltpu.sync_copy(x_hbm.at[pl.ds(row0, R15)],
                        acc_sh.at[pl.ds(row0, R15)])

    @pl.when(jnp.logical_and(cid != 0, jnp.logical_not(last)))
    def _():
        for t in range(4):
            pltpu.sync_copy(rows, acc_sh.at[pl.ds(row0 + t * K, K)])
        pltpu.sync_copy(rows.at[pl.ds(0, 112)],
                        acc_sh.at[pl.ds(row0 + 4 * K, 112)])

    @pl.when(jnp.logical_and(cid != 0, last))
    def _():
        for t in range(5):
            pltpu.sync_copy(rows, acc_sh.at[pl.ds(row0 + t * K, K)])

    pltpu.sync_copy(src_hbm.at[wid], sidx)
    pltpu.sync_copy(dst_hbm.at[wid], didx)
    plsc.subcore_barrier()

    def chunk(j, carry):
        pltpu.async_copy(x_hbm.at[sidx.at[j]], rows, sem).wait()
        pltpu.sync_copy(rows, acc_sh.at[didx.at[j]], add=True)
        return carry
    lax.fori_loop(0, NCHUNK, chunk, 0)
    plsc.subcore_barrier()

    @pl.when(jnp.logical_not(last))
    def _():
        pltpu.sync_copy(acc_sh.at[pl.ds(row0, R0)],
                        out_hbm.at[cid, pl.ds(row0, R0)])

    @pl.when(last)
    def _():
        pltpu.sync_copy(acc_sh.at[pl.ds(row0, R15)],
                        out_hbm.at[cid, pl.ds(row0, R15)])


@functools.partial(
    pl.kernel, mesh=_MESH,
    out_type=[jax.ShapeDtypeStruct((NC, N, DEGW), jnp.float32)],
    scratch_types=[
        pltpu.VMEM_SHARED((N + 16, DEGW), jnp.float32),
        pltpu.VMEM((NCHUNK, K), jnp.int32),
        pltpu.VMEM((K, DEGW), jnp.float32),
        pltpu.VMEM((16, DEGW), jnp.float32),
        pltpu.SemaphoreType.DMA,
    ])
def _sc_deg(dst_hbm, degout_hbm, deg_sh, didx, ones, zdbuf, sem):
    cid = lax.axis_index("c")
    sid = lax.axis_index("s")
    wid = cid * NS + sid
    row0 = sid * R0
    last = sid == NS - 1

    def orow(r, carry):
        for j in range(DEGW // 16):
            ones[r, pl.ds(j * 16, 16)] = jnp.full((16,), 1.0, jnp.float32)
        return carry
    lax.fori_loop(0, K, orow, 0)

    def zrow(r, carry):
        for j in range(DEGW // 16):
            zdbuf[r, pl.ds(j * 16, 16)] = jnp.zeros((16,), jnp.float32)
        return carry
    lax.fori_loop(0, 16, zrow, 0)

    @pl.when(jnp.logical_not(last))
    def _():
        for t in range(R0 // 16):
            pltpu.sync_copy(zdbuf, deg_sh.at[pl.ds(row0 + t * 16, 16)])

    @pl.when(last)
    def _():
        for t in range(R15 // 16):
            pltpu.sync_copy(zdbuf, deg_sh.at[pl.ds(row0 + t * 16, 16)])

    pltpu.sync_copy(dst_hbm.at[wid], didx)
    plsc.subcore_barrier()

    def chunk(j, carry):
        pltpu.sync_copy(ones, deg_sh.at[didx.at[j]], add=True)
        return carry
    lax.fori_loop(0, NCHUNK, chunk, 0)
    plsc.subcore_barrier()

    @pl.when(jnp.logical_not(last))
    def _():
        pltpu.sync_copy(deg_sh.at[pl.ds(row0, R0)],
                        degout_hbm.at[cid, pl.ds(row0, R0)])

    @pl.when(last)
    def _():
        pltpu.sync_copy(deg_sh.at[pl.ds(row0, R15)],
                        degout_hbm.at[cid, pl.ds(row0, R15)])


RB = 2000


def _make_tc_layer(relu: bool):
    def body(p_ref, d_ref, w_ref, b_ref, o_ref):
        num = p_ref[0] + p_ref[1]
        deg = d_ref[0, :, :1] + d_ref[1, :, :1] + 1.0
        h = num / deg
        out = jnp.dot(h, w_ref[...], preferred_element_type=jnp.float32)
        out = out + b_ref[...]
        if relu:
            out = jnp.maximum(out, 0.0)
        o_ref[...] = out

    return pl.pallas_call(
        body,
        grid=(N // RB,),
        in_specs=[
            pl.BlockSpec((NC, RB, D), lambda i: (0, i, 0)),
            pl.BlockSpec((NC, RB, DEGW), lambda i: (0, i, 0)),
            pl.BlockSpec((D, D), lambda i: (0, 0)),
            pl.BlockSpec((1, D), lambda i: (0, 0)),
        ],
        out_specs=pl.BlockSpec((RB, D), lambda i: (i, 0)),
        out_shape=jax.ShapeDtypeStruct((N, D), jnp.float32),
    )


_tc_relu = _make_tc_layer(relu=True)
_tc_lin = _make_tc_layer(relu=False)


def kernel(g, features, W1, b1, W2, b2):
    pad = EPAD - EPW
    src = jnp.pad(g[0].reshape(NW, EPW), ((0, 0), (0, pad)),
                  constant_values=0).reshape(NW, NCHUNK, K)
    dst = jnp.pad(g[1].reshape(NW, EPW), ((0, 0), (0, pad)),
                  constant_values=N).reshape(NW, NCHUNK, K)
    (degp,) = _sc_deg(dst)
    (part1,) = _sc_agg(features, src, dst)
    h1 = _tc_relu(part1, degp, W1, b1.reshape(1, D))
    (part2,) = _sc_agg(h1, src, dst)
    out = _tc_lin(part2, degp, W2, b2.reshape(1, D))
    return out


# speedup vs baseline: 10.5532x; 2.0165x over previous
import functools

import jax
import jax.numpy as jnp
from jax import lax
from jax.experimental import pallas as pl
from jax.experimental.pallas import tpu as pltpu
from jax.experimental.pallas import tpu_sc as plsc

N = 10000
E = 320000
D = 128

NC = 2
NS = 16
NW = NC * NS
EPW = E // NW
K = 80
NCHUNK = EPW // K
R0 = 624
R15 = N - 15 * R0
DEGW = 128

_MESH = plsc.VectorSubcoreMesh(core_axis_name="c", subcore_axis_name="s")


@functools.partial(
    pl.kernel, mesh=_MESH,
    out_type=[jax.ShapeDtypeStruct((NC, N, D), jnp.float32)],
    scratch_types=[
        pltpu.VMEM_SHARED((N, D), jnp.float32),
        pltpu.VMEM((EPW,), jnp.int32),
        pltpu.VMEM((NCHUNK, K), jnp.int32),
        pltpu.VMEM((2, K, D), jnp.float32),
        pltpu.SemaphoreType.DMA,
        pltpu.SemaphoreType.DMA,
    ])
def _sc_agg(x_hbm, src_hbm, dst_hbm, out_hbm, acc_sh, sidx, didx, rows,
            sem0, sem1):
    cid = lax.axis_index("c")
    sid = lax.axis_index("s")
    wid = cid * NS + sid
    row0 = sid * R0
    last = sid == NS - 1

    def _gather(j, b, sem):
        return pltpu.make_async_copy(
            x_hbm.at[sidx.at[pl.ds(j * K, K)]], rows.at[b], sem)

    pltpu.sync_copy(src_hbm.at[pl.ds(wid * EPW, EPW)], sidx)
    pltpu.sync_copy(dst_hbm.at[wid], didx)

    def zrow(r, carry):
        for j in range(D // 16):
            rows[0, r, pl.ds(j * 16, 16)] = jnp.zeros((16,), jnp.float32)
        return carry
    lax.fori_loop(0, K, zrow, 0)

    @pl.when(jnp.logical_and(cid == 0, jnp.logical_not(last)))
    def _():
        pltpu.sync_copy(x_hbm.at[pl.ds(row0, R0)],
                        acc_sh.at[pl.ds(row0, R0)])

    @pl.when(jnp.logical_and(cid == 0, last))
    def _():
        pltpu.sync_copy(x_hbm.at[pl.ds(row0, R15)],
                        acc_sh.at[pl.ds(row0, R15)])

    @pl.when(jnp.logical_and(cid != 0, jnp.logical_not(last)))
    def _():
        for t in range(7):
            pltpu.sync_copy(rows.at[0],
                            acc_sh.at[pl.ds(row0 + t * K, K)])
        pltpu.sync_copy(rows.at[0].at[pl.ds(0, 64)],
                        acc_sh.at[pl.ds(row0 + 7 * K, 64)])

    @pl.when(jnp.logical_and(cid != 0, last))
    def _():
        for t in range(8):
            pltpu.sync_copy(rows.at[0],
                            acc_sh.at[pl.ds(row0 + t * K, K)])

    plsc.subcore_barrier()

    _gather(0, 0, sem0).start()

    def body(t, carry):
        j0 = 2 * t
        _gather(j0 + 1, 1, sem1).start()
        _gather(j0, 0, sem0).wait()
        pltpu.sync_copy(rows.at[0], acc_sh.at[didx.at[j0]], add=True)
        _gather(j0 + 2, 0, sem0).start()
        _gather(j0 + 1, 1, sem1).wait()
        pltpu.sync_copy(rows.at[1], acc_sh.at[didx.at[j0 + 1]], add=True)
        return carry
    lax.fori_loop(0, (NCHUNK - 1) // 2, body, 0)
    _gather(NCHUNK - 1, 0, sem0).wait()
    pltpu.sync_copy(rows.at[0], acc_sh.at[didx.at[NCHUNK - 1]], add=True)
    plsc.subcore_barrier()

    @pl.when(jnp.logical_not(last))
    def _():
        pltpu.sync_copy(acc_sh.at[pl.ds(row0, R0)],
                        out_hbm.at[cid, pl.ds(row0, R0)])

    @pl.when(last)
    def _():
        pltpu.sync_copy(acc_sh.at[pl.ds(row0, R15)],
                        out_hbm.at[cid, pl.ds(row0, R15)])


@functools.partial(
    pl.kernel, mesh=_MESH,
    out_type=[jax.ShapeDtypeStruct((NC, N, DEGW), jnp.float32)],
    scratch_types=[
        pltpu.VMEM_SHARED((N, DEGW), jnp.float32),
        pltpu.VMEM((NCHUNK, K), jnp.int32),
        pltpu.VMEM((K, DEGW), jnp.float32),
        pltpu.VMEM((16, DEGW), jnp.float32),
        pltpu.SemaphoreType.DMA,
    ])
def _sc_deg(dst_hbm, degout_hbm, deg_sh, didx, ones, zdbuf, sem):
    cid = lax.axis_index("c")
    sid = lax.axis_index("s")
    wid = cid * NS + sid
    row0 = sid * R0
    last = sid == NS - 1

    def orow(r, carry):
        for j in range(DEGW // 16):
            ones[r, pl.ds(j * 16, 16)] = jnp.full((16,), 1.0, jnp.float32)
        return carry
    lax.fori_loop(0, K, orow, 0)

    def zrow(r, carry):
        for j in range(DEGW // 16):
            zdbuf[r, pl.ds(j * 16, 16)] = jnp.zeros((16,), jnp.float32)
        return carry
    lax.fori_loop(0, 16, zrow, 0)

    @pl.when(jnp.logical_not(last))
    def _():
        for t in range(R0 // 16):
            pltpu.sync_copy(zdbuf, deg_sh.at[pl.ds(row0 + t * 16, 16)])

    @pl.when(last)
    def _():
        for t in range(R15 // 16):
            pltpu.sync_copy(zdbuf, deg_sh.at[pl.ds(row0 + t * 16, 16)])

    pltpu.sync_copy(dst_hbm.at[wid], didx)
    plsc.subcore_barrier()

    def fire(j, carry):
        pltpu.async_copy(ones, deg_sh.at[didx.at[j]], sem, add=True)
        return carry
    lax.fori_loop(0, NCHUNK, fire, 0)

    def drain(j, carry):
        pltpu.make_async_copy(ones, deg_sh.at[didx.at[j]], sem).wait()
        return carry
    lax.fori_loop(0, NCHUNK, drain, 0)
    plsc.subcore_barrier()

    @pl.when(jnp.logical_not(last))
    def _():
        pltpu.sync_copy(deg_sh.at[pl.ds(row0, R0)],
                        degout_hbm.at[cid, pl.ds(row0, R0)])

    @pl.when(last)
    def _():
        pltpu.sync_copy(deg_sh.at[pl.ds(row0, R15)],
                        degout_hbm.at[cid, pl.ds(row0, R15)])


RB = 2000


def _make_tc_layer(relu: bool):
    def body(p_ref, d_ref, w_ref, b_ref, o_ref):
        num = p_ref[0] + p_ref[1]
        deg = d_ref[0, :, :1] + d_ref[1, :, :1] + 1.0
        h = num / deg
        out = jnp.dot(h, w_ref[...], preferred_element_type=jnp.float32)
        out = out + b_ref[...]
        if relu:
            out = jnp.maximum(out, 0.0)
        o_ref[...] = out

    return pl.pallas_call(
        body,
        grid=(N // RB,),
        in_specs=[
            pl.BlockSpec((NC, RB, D), lambda i: (0, i, 0)),
            pl.BlockSpec((NC, RB, DEGW), lambda i: (0, i, 0)),
            pl.BlockSpec((D, D), lambda i: (0, 0)),
            pl.BlockSpec((1, D), lambda i: (0, 0)),
        ],
        out_specs=pl.BlockSpec((RB, D), lambda i: (i, 0)),
        out_shape=jax.ShapeDtypeStruct((N, D), jnp.float32),
    )


_tc_relu = _make_tc_layer(relu=True)
_tc_lin = _make_tc_layer(relu=False)


def kernel(g, features, W1, b1, W2, b2):
    src = g[0]
    dst = g[1].reshape(NW, NCHUNK, K)
    (degp,) = _sc_deg(dst)
    (part1,) = _sc_agg(features, src, dst)
    h1 = _tc_relu(part1, degp, W1, b1.reshape(1, D))
    (part2,) = _sc_agg(h1, src, dst)
    out = _tc_lin(part2, degp, W2, b2.reshape(1, D))
    return out


# speedup vs baseline: 12.0216x; 1.1391x over previous
import functools

import jax
import jax.numpy as jnp
from jax import lax
from jax.experimental import pallas as pl
from jax.experimental.pallas import tpu as pltpu
from jax.experimental.pallas import tpu_sc as plsc

N = 10000
E = 320000
D = 128

NC = 2
NS = 16
NW = NC * NS
EPW = E // NW
K = 80
NCHUNK = EPW // K
R0 = 624
R15 = N - 15 * R0
DEGW = 128

_MESH = plsc.VectorSubcoreMesh(core_axis_name="c", subcore_axis_name="s")


@functools.partial(
    pl.kernel, mesh=_MESH,
    out_type=[jax.ShapeDtypeStruct((NC, N, D), jnp.float32)],
    scratch_types=[
        pltpu.VMEM_SHARED((N, D), jnp.float32),
        pltpu.VMEM((EPW,), jnp.int32),
        pltpu.VMEM((EPW,), jnp.int32),
        pltpu.VMEM((3, K, D), jnp.float32),
        pltpu.SemaphoreType.DMA,
        pltpu.SemaphoreType.DMA,
        pltpu.SemaphoreType.DMA,
    ])
def _sc_agg(x_hbm, src_hbm, dst_hbm, out_hbm, acc_sh, sidx, didx, rows,
            sem0, sem1, sem2):
    cid = lax.axis_index("c")
    sid = lax.axis_index("s")
    wid = cid * NS + sid
    row0 = sid * R0
    last = sid == NS - 1
    sems = (sem0, sem1, sem2)

    def _gather(j, b):
        return pltpu.make_async_copy(
            x_hbm.at[sidx.at[pl.ds(j * K, K)]], rows.at[b], sems[b])

    pltpu.sync_copy(src_hbm.at[pl.ds(wid * EPW, EPW)], sidx)
    pltpu.sync_copy(dst_hbm.at[pl.ds(wid * EPW, EPW)], didx)

    def zrow(r, carry):
        for j in range(D // 16):
            rows[0, r, pl.ds(j * 16, 16)] = jnp.zeros((16,), jnp.float32)
        return carry
    lax.fori_loop(0, K, zrow, 0)

    @pl.when(jnp.logical_and(cid == 0, jnp.logical_not(last)))
    def _():
        pltpu.sync_copy(x_hbm.at[pl.ds(row0, R0)],
                        acc_sh.at[pl.ds(row0, R0)])

    @pl.when(jnp.logical_and(cid == 0, last))
    def _():
        pltpu.sync_copy(x_hbm.at[pl.ds(row0, R15)],
                        acc_sh.at[pl.ds(row0, R15)])

    @pl.when(jnp.logical_and(cid != 0, jnp.logical_not(last)))
    def _():
        for t in range(7):
            pltpu.sync_copy(rows.at[0],
                            acc_sh.at[pl.ds(row0 + t * K, K)])
        pltpu.sync_copy(rows.at[0].at[pl.ds(0, 64)],
                        acc_sh.at[pl.ds(row0 + 7 * K, 64)])

    @pl.when(jnp.logical_and(cid != 0, last))
    def _():
        for t in range(8):
            pltpu.sync_copy(rows.at[0],
                            acc_sh.at[pl.ds(row0 + t * K, K)])

    plsc.subcore_barrier()

    _gather(0, 0).start()
    _gather(1, 1).start()

    def body(t, carry):
        for o in range(3):
            j = 3 * t + o
            _gather(j + 2, (o + 2) % 3).start()
            _gather(j, o).wait()
            pltpu.sync_copy(rows.at[o],
                            acc_sh.at[didx.at[pl.ds(j * K, K)]], add=True)
        return carry
    lax.fori_loop(0, (NCHUNK - 2) // 3, body, 0)
    _gather(NCHUNK - 2, 0).wait()
    pltpu.sync_copy(rows.at[0],
                    acc_sh.at[didx.at[pl.ds((NCHUNK - 2) * K, K)]], add=True)
    _gather(NCHUNK - 1, 1).wait()
    pltpu.sync_copy(rows.at[1],
                    acc_sh.at[didx.at[pl.ds((NCHUNK - 1) * K, K)]], add=True)
    plsc.subcore_barrier()

    @pl.when(jnp.logical_not(last))
    def _():
        pltpu.sync_copy(acc_sh.at[pl.ds(row0, R0)],
                        out_hbm.at[cid, pl.ds(row0, R0)])

    @pl.when(last)
    def _():
        pltpu.sync_copy(acc_sh.at[pl.ds(row0, R15)],
                        out_hbm.at[cid, pl.ds(row0, R15)])


@functools.partial(
    pl.kernel, mesh=_MESH,
    out_type=[jax.ShapeDtypeStruct((NC, N, DEGW), jnp.float32)],
    scratch_types=[
        pltpu.VMEM_SHARED((N, DEGW), jnp.float32),
        pltpu.VMEM((NCHUNK, K), jnp.int32),
        pltpu.VMEM((K, DEGW), jnp.float32),
        pltpu.VMEM((16, DEGW), jnp.float32),
        pltpu.SemaphoreType.DMA,
    ])
def _sc_deg(dst_hbm, degout_hbm, deg_sh, didx, ones, zdbuf, sem):
    cid = lax.axis_index("c")
    sid = lax.axis_index("s")
    wid = cid * NS + sid
    row0 = sid * R0
    last = sid == NS - 1

    def orow(r, carry):
        for j in range(DEGW // 16):
            ones[r, pl.ds(j * 16, 16)] = jnp.full((16,), 1.0, jnp.float32)
        return carry
    lax.fori_loop(0, K, orow, 0)

    def zrow(r, carry):
        for j in range(DEGW // 16):
            zdbuf[r, pl.ds(j * 16, 16)] = jnp.zeros((16,), jnp.float32)
        return carry
    lax.fori_loop(0, 16, zrow, 0)

    @pl.when(jnp.logical_not(last))
    def _():
        for t in range(R0 // 16):
            pltpu.sync_copy(zdbuf, deg_sh.at[pl.ds(row0 + t * 16, 16)])

    @pl.when(last)
    def _():
        for t in range(R15 // 16):
            pltpu.sync_copy(zdbuf, deg_sh.at[pl.ds(row0 + t * 16, 16)])

    pltpu.sync_copy(dst_hbm.at[wid], didx)
    plsc.subcore_barrier()

    def fire(j, carry):
        pltpu.async_copy(ones, deg_sh.at[didx.at[j]], sem, add=True)
        return carry
    lax.fori_loop(0, NCHUNK, fire, 0)

    def drain(j, carry):
        pltpu.make_async_copy(ones, deg_sh.at[didx.at[j]], sem).wait()
        return carry
    lax.fori_loop(0, NCHUNK, drain, 0)
    plsc.subcore_barrier()

    @pl.when(jnp.logical_not(last))
    def _():
        pltpu.sync_copy(deg_sh.at[pl.ds(row0, R0)],
                        degout_hbm.at[cid, pl.ds(row0, R0)])

    @pl.when(last)
    def _():
        pltpu.sync_copy(deg_sh.at[pl.ds(row0, R15)],
                        degout_hbm.at[cid, pl.ds(row0, R15)])


RB = 2000


def _make_tc_layer(relu: bool):
    def body(p_ref, d_ref, w_ref, b_ref, o_ref):
        num = p_ref[0] + p_ref[1]
        deg = d_ref[0, :, :1] + d_ref[1, :, :1] + 1.0
        h = num / deg
        out = jnp.dot(h, w_ref[...], preferred_element_type=jnp.float32)
        out = out + b_ref[...]
        if relu:
            out = jnp.maximum(out, 0.0)
        o_ref[...] = out

    return pl.pallas_call(
        body,
        grid=(N // RB,),
        in_specs=[
            pl.BlockSpec((NC, RB, D), lambda i: (0, i, 0)),
            pl.BlockSpec((NC, RB, DEGW), lambda i: (0, i, 0)),
            pl.BlockSpec((D, D), lambda i: (0, 0)),
            pl.BlockSpec((1, D), lambda i: (0, 0)),
        ],
        out_specs=pl.BlockSpec((RB, D), lambda i: (i, 0)),
        out_shape=jax.ShapeDtypeStruct((N, D), jnp.float32),
    )


_tc_relu = _make_tc_layer(relu=True)
_tc_lin = _make_tc_layer(relu=False)


def kernel(g, features, W1, b1, W2, b2):
    src = g[0]
    dst = g[1]
    dst3 = dst.reshape(NW, NCHUNK, K)
    (degp,) = _sc_deg(dst3)
    (part1,) = _sc_agg(features, src, dst)
    h1 = _tc_relu(part1, degp, W1, b1.reshape(1, D))
    (part2,) = _sc_agg(h1, src, dst)
    out = _tc_lin(part2, degp, W2, b2.reshape(1, D))
    return out
